# initial kernel scaffold (unmeasured)
import jax
import jax.numpy as jnp
from jax import lax
from jax.experimental import pallas as pl
from jax.experimental.pallas import tpu as pltpu

N_DEV = 32


def kernel(A, B):
    m, k_loc = A.shape
    _, n = B.shape
    chunk = m // N_DEV

    def body(a_ref, b_ref, out_ref, gather_ref,
             send1_sems, recv1_sems, send2_sems, recv2_sems):
        my = lax.axis_index("i")

        out_ref[:, :] = jnp.dot(
            a_ref[:, :], b_ref[:, :], preferred_element_type=jnp.float32
        )

        def p1_desc(j):
            return pltpu.make_async_remote_copy(
                src_ref=out_ref.at[pl.ds(j * chunk, chunk), :],
                dst_ref=gather_ref.at[my],
                send_sem=send1_sems.at[j],
                recv_sem=recv1_sems.at[my],
                device_id=(j,),
                device_id_type=pl.DeviceIdType.MESH,
            )

        for j in range(N_DEV):
            @pl.when(my != j)
            def _(j=j):
                p1_desc(j).start()

        for j in range(N_DEV):
            @pl.when(my == j)
            def _(j=j):
                gather_ref[j, :, :] = out_ref[pl.ds(j * chunk, chunk), :]

        for j in range(N_DEV):
            @pl.when(my != j)
            def _(j=j):
                pltpu.make_async_remote_copy(
                    src_ref=gather_ref.at[j],
                    dst_ref=gather_ref.at[j],
                    send_sem=send1_sems.at[j],
                    recv_sem=recv1_sems.at[j],
                    device_id=(j,),
                    device_id_type=pl.DeviceIdType.MESH,
                ).wait_recv()

        for j in range(N_DEV):
            @pl.when(my != j)
            def _(j=j):
                p1_desc(j).wait_send()

        reduced = jnp.sum(gather_ref[:, :, :], axis=0)
        for j in range(N_DEV):
            @pl.when(my == j)
            def _(j=j):
                out_ref[pl.ds(j * chunk, chunk), :] = reduced

        def p2_desc(j):
            return pltpu.make_async_remote_copy(
                src_ref=out_ref.at[pl.ds(my * chunk, chunk), :],
                dst_ref=out_ref.at[pl.ds(my * chunk, chunk), :],
                send_sem=send2_sems.at[j],
                recv_sem=recv2_sems.at[my],
                device_id=(j,),
                device_id_type=pl.DeviceIdType.MESH,
            )

        for j in range(N_DEV):
            @pl.when(my != j)
            def _(j=j):
                p2_desc(j).start()

        for j in range(N_DEV):
            @pl.when(my != j)
            def _(j=j):
                pltpu.make_async_remote_copy(
                    src_ref=out_ref.at[pl.ds(j * chunk, chunk), :],
                    dst_ref=out_ref.at[pl.ds(j * chunk, chunk), :],
                    send_sem=send2_sems.at[j],
                    recv_sem=recv2_sems.at[j],
                    device_id=(j,),
                    device_id_type=pl.DeviceIdType.MESH,
                ).wait_recv()

        for j in range(N_DEV):
            @pl.when(my != j)
            def _(j=j):
                p2_desc(j).wait_send()

    return pl.pallas_call(
        body,
        out_shape=jax.ShapeDtypeStruct((m, n), jnp.float32),
        in_specs=[
            pl.BlockSpec(memory_space=pltpu.VMEM),
            pl.BlockSpec(memory_space=pltpu.VMEM),
        ],
        out_specs=pl.BlockSpec(memory_space=pltpu.VMEM),
        scratch_shapes=[
            pltpu.VMEM((N_DEV, chunk, n), jnp.float32),
            pltpu.SemaphoreType.DMA((N_DEV,)),
            pltpu.SemaphoreType.DMA((N_DEV,)),
            pltpu.SemaphoreType.DMA((N_DEV,)),
            pltpu.SemaphoreType.DMA((N_DEV,)),
        ],
        compiler_params=pltpu.CompilerParams(collective_id=0),
    )(A, B)


# baseline (device time: 279171 ns/iter reference)
import jax
import jax.numpy as jnp
from jax import lax
from jax.experimental import pallas as pl
from jax.experimental.pallas import tpu as pltpu

N_DEV = 32


def kernel(A, B):
    m, k_loc = A.shape
    _, n = B.shape
    chunk = m // N_DEV

    def body(a_ref, b_ref, out_ref, gather_ref,
             send1_sems, recv1_sems, send2_sems, recv2_sems):
        my = lax.axis_index("i")

        out_ref[:, :] = jnp.dot(
            a_ref[:, :], b_ref[:, :], preferred_element_type=jnp.float32
        )

        def p1_desc(j):
            return pltpu.make_async_remote_copy(
                src_ref=out_ref.at[pl.ds(j * chunk, chunk), :],
                dst_ref=gather_ref.at[my],
                send_sem=send1_sems.at[j],
                recv_sem=recv1_sems.at[my],
                device_id=(j,),
                device_id_type=pl.DeviceIdType.MESH,
            )

        for j in range(N_DEV):
            @pl.when(my != j)
            def _(j=j):
                p1_desc(j).start()

        for j in range(N_DEV):
            @pl.when(my == j)
            def _(j=j):
                gather_ref[j, :, :] = out_ref[pl.ds(j * chunk, chunk), :]

        for j in range(N_DEV):
            @pl.when(my != j)
            def _(j=j):
                pltpu.make_async_remote_copy(
                    src_ref=gather_ref.at[j],
                    dst_ref=gather_ref.at[j],
                    send_sem=send1_sems.at[j],
                    recv_sem=recv1_sems.at[j],
                    device_id=(j,),
                    device_id_type=pl.DeviceIdType.MESH,
                ).wait_recv()

        for j in range(N_DEV):
            @pl.when(my != j)
            def _(j=j):
                p1_desc(j).wait_send()

        reduced = jnp.sum(gather_ref[:, :, :], axis=0)
        for j in range(N_DEV):
            @pl.when(my == j)
            def _(j=j):
                out_ref[pl.ds(j * chunk, chunk), :] = reduced

        def p2_desc(j):
            return pltpu.make_async_remote_copy(
                src_ref=out_ref.at[pl.ds(my * chunk, chunk), :],
                dst_ref=out_ref.at[pl.ds(my * chunk, chunk), :],
                send_sem=send2_sems.at[j],
                recv_sem=recv2_sems.at[my],
                device_id=(j,),
                device_id_type=pl.DeviceIdType.MESH,
            )

        for j in range(N_DEV):
            @pl.when(my != j)
            def _(j=j):
                p2_desc(j).start()

        for j in range(N_DEV):
            @pl.when(my != j)
            def _(j=j):
                pltpu.make_async_remote_copy(
                    src_ref=out_ref.at[pl.ds(j * chunk, chunk), :],
                    dst_ref=out_ref.at[pl.ds(j * chunk, chunk), :],
                    send_sem=send2_sems.at[j],
                    recv_sem=recv2_sems.at[j],
                    device_id=(j,),
                    device_id_type=pl.DeviceIdType.MESH,
                ).wait_recv()

        for j in range(N_DEV):
            @pl.when(my != j)
            def _(j=j):
                p2_desc(j).wait_send()

    return pl.pallas_call(
        body,
        out_shape=jax.ShapeDtypeStruct((m, n), jnp.float32),
        in_specs=[
            pl.BlockSpec(memory_space=pltpu.VMEM),
            pl.BlockSpec(memory_space=pltpu.VMEM),
        ],
        out_specs=pl.BlockSpec(memory_space=pltpu.VMEM),
        scratch_shapes=[
            pltpu.VMEM((N_DEV, chunk, n), jnp.float32),
            pltpu.SemaphoreType.DMA((N_DEV,)),
            pltpu.SemaphoreType.DMA((N_DEV,)),
            pltpu.SemaphoreType.DMA((N_DEV,)),
            pltpu.SemaphoreType.DMA((N_DEV,)),
        ],
    )(A, B)


# device time: 146358 ns/iter; 1.9075x vs baseline; 1.9075x over previous
import jax
import jax.numpy as jnp
from jax import lax
from jax.experimental import pallas as pl
from jax.experimental.pallas import tpu as pltpu

N_DEV = 32


def kernel(A, B):
    m, k_loc = A.shape
    _, n = B.shape
    chunk = m // N_DEV

    def body(a_ref, b_ref, out_ref, stage_ref, gather_ref, bcast_ref,
             red_ref, s1, r1, s2, r2):
        my = lax.axis_index("i")

        out_ref[:, :] = jnp.dot(
            a_ref[:, :].astype(jnp.bfloat16),
            b_ref[:, :].astype(jnp.bfloat16),
            preferred_element_type=jnp.float32,
        )

        for d in range(1, N_DEV):
            t = lax.rem(my + d, N_DEV)
            stage_ref[d, :, :] = out_ref[pl.ds(t * chunk, chunk), :].astype(
                jnp.bfloat16
            )
        gather_ref[0, :, :] = out_ref[pl.ds(my * chunk, chunk), :].astype(
            jnp.bfloat16
        )

        def p1(d):
            t = lax.rem(my + d, N_DEV)
            return pltpu.make_async_remote_copy(
                src_ref=stage_ref.at[d],
                dst_ref=gather_ref.at[d],
                send_sem=s1.at[d],
                recv_sem=r1.at[d],
                device_id=(t,),
                device_id_type=pl.DeviceIdType.MESH,
            )

        for d in range(1, N_DEV):
            p1(d).start()
        for d in range(1, N_DEV):
            p1(d).wait_recv()

        reduced = jnp.sum(gather_ref[:, :, :].astype(jnp.float32), axis=0)
        red_ref[:, :] = reduced.astype(jnp.bfloat16)

        def p2(d):
            t = lax.rem(my + d, N_DEV)
            return pltpu.make_async_remote_copy(
                src_ref=red_ref,
                dst_ref=bcast_ref.at[d],
                send_sem=s2.at[d],
                recv_sem=r2.at[d],
                device_id=(t,),
                device_id_type=pl.DeviceIdType.MESH,
            )

        for d in range(1, N_DEV):
            p2(d).start()

        for d in range(1, N_DEV):
            p1(d).wait_send()

        out_ref[pl.ds(my * chunk, chunk), :] = reduced

        for d in range(1, N_DEV):
            p2(d).wait_recv()
            s = lax.rem(my - d + N_DEV, N_DEV)
            out_ref[pl.ds(s * chunk, chunk), :] = bcast_ref[d, :, :].astype(
                jnp.float32
            )

        for d in range(1, N_DEV):
            p2(d).wait_send()

    cdt = jnp.bfloat16
    return pl.pallas_call(
        body,
        out_shape=jax.ShapeDtypeStruct((m, n), jnp.float32),
        in_specs=[
            pl.BlockSpec(memory_space=pltpu.VMEM),
            pl.BlockSpec(memory_space=pltpu.VMEM),
        ],
        out_specs=pl.BlockSpec(memory_space=pltpu.VMEM),
        scratch_shapes=[
            pltpu.VMEM((N_DEV, chunk, n), cdt),
            pltpu.VMEM((N_DEV, chunk, n), cdt),
            pltpu.VMEM((N_DEV, chunk, n), cdt),
            pltpu.VMEM((chunk, n), cdt),
            pltpu.SemaphoreType.DMA((N_DEV,)),
            pltpu.SemaphoreType.DMA((N_DEV,)),
            pltpu.SemaphoreType.DMA((N_DEV,)),
            pltpu.SemaphoreType.DMA((N_DEV,)),
        ],
    )(A, B)


# device time: 137853 ns/iter; 2.0251x vs baseline; 1.0617x over previous
import jax
import jax.numpy as jnp
from jax import lax
from jax.experimental import pallas as pl
from jax.experimental.pallas import tpu as pltpu

N_DEV = 32


def kernel(A, B):
    m, k_loc = A.shape
    _, n = B.shape
    chunk = m // N_DEV

    sub = chunk // 2

    def body(a_ref, b_ref, out_ref, stage_ref, gather_ref, bcast_ref,
             red_ref, s1, r1, s2, r2):
        my = lax.axis_index("i")

        out_ref[:, :] = jnp.dot(
            a_ref[:, :].astype(jnp.bfloat16),
            b_ref[:, :].astype(jnp.bfloat16),
            preferred_element_type=jnp.float32,
        )

        for d in range(1, N_DEV):
            t = lax.rem(my + d, N_DEV)
            stage_ref[d, :, :] = out_ref[pl.ds(t * chunk, chunk), :].astype(
                jnp.bfloat16
            )
        gather_ref[0, :, :] = out_ref[pl.ds(my * chunk, chunk), :].astype(
            jnp.bfloat16
        )

        def p1(c, d):
            t = lax.rem(my + d, N_DEV)
            return pltpu.make_async_remote_copy(
                src_ref=stage_ref.at[d, pl.ds(c * sub, sub), :],
                dst_ref=gather_ref.at[d, pl.ds(c * sub, sub), :],
                send_sem=s1.at[c, d],
                recv_sem=r1.at[c, d],
                device_id=(t,),
                device_id_type=pl.DeviceIdType.MESH,
            )

        def p2(c, d):
            t = lax.rem(my + d, N_DEV)
            return pltpu.make_async_remote_copy(
                src_ref=red_ref.at[pl.ds(c * sub, sub), :],
                dst_ref=bcast_ref.at[d, pl.ds(c * sub, sub), :],
                send_sem=s2.at[c, d],
                recv_sem=r2.at[c, d],
                device_id=(t,),
                device_id_type=pl.DeviceIdType.MESH,
            )

        for c in range(2):
            for d in range(1, N_DEV):
                p1(c, d).start()

        for c in range(2):
            for d in range(1, N_DEV):
                p1(c, d).wait_recv()
            reduced_c = jnp.sum(
                gather_ref[:, c * sub:(c + 1) * sub, :].astype(jnp.float32),
                axis=0,
            )
            red_ref[pl.ds(c * sub, sub), :] = reduced_c.astype(jnp.bfloat16)
            out_ref[pl.ds(my * chunk + c * sub, sub), :] = reduced_c
            for d in range(1, N_DEV):
                p2(c, d).start()

        for c in range(2):
            for d in range(1, N_DEV):
                p1(c, d).wait_send()

        for c in range(2):
            for d in range(1, N_DEV):
                p2(c, d).wait_recv()
                s = lax.rem(my - d + N_DEV, N_DEV)
                out_ref[pl.ds(s * chunk + c * sub, sub), :] = bcast_ref[
                    d, pl.ds(c * sub, sub), :
                ].astype(jnp.float32)

        for c in range(2):
            for d in range(1, N_DEV):
                p2(c, d).wait_send()

    cdt = jnp.bfloat16
    return pl.pallas_call(
        body,
        out_shape=jax.ShapeDtypeStruct((m, n), jnp.float32),
        in_specs=[
            pl.BlockSpec(memory_space=pltpu.VMEM),
            pl.BlockSpec(memory_space=pltpu.VMEM),
        ],
        out_specs=pl.BlockSpec(memory_space=pltpu.VMEM),
        scratch_shapes=[
            pltpu.VMEM((N_DEV, chunk, n), cdt),
            pltpu.VMEM((N_DEV, chunk, n), cdt),
            pltpu.VMEM((N_DEV, chunk, n), cdt),
            pltpu.VMEM((chunk, n), cdt),
            pltpu.SemaphoreType.DMA((2, N_DEV)),
            pltpu.SemaphoreType.DMA((2, N_DEV)),
            pltpu.SemaphoreType.DMA((2, N_DEV)),
            pltpu.SemaphoreType.DMA((2, N_DEV)),
        ],
    )(A, B)
